# Initial kernel scaffold; baseline (speedup 1.0000x reference)
import jax
import jax.numpy as jnp
from jax.experimental import pallas as pl


def kernel(x, base_W, base_b, A, B, router_W):
    raise NotImplementedError("write your pallas kernel here")


# speedup vs baseline: 3.8710x; 3.8710x over previous
import jax
import jax.numpy as jnp
from jax.experimental import pallas as pl

SCALING = 32.0 / 64.0


def _fused_kernel(x_ref, w_ref, a_ref, r_ref, bflat_ref, bias_ref, out_ref):
    xb = x_ref[...]
    base = jnp.dot(xb, w_ref[...], preferred_element_type=jnp.float32)
    h = jnp.dot(xb, a_ref[...], preferred_element_type=jnp.float32)
    h = jnp.maximum(h, 0.0)
    logits = jnp.dot(xb, r_ref[...], preferred_element_type=jnp.float32)
    m = jnp.max(logits, axis=-1, keepdims=True)
    p = jnp.exp(logits - m)
    wts = p / jnp.sum(p, axis=-1, keepdims=True)
    num_e = wts.shape[-1]
    hw = jnp.concatenate(
        [wts[:, e:e + 1] * h for e in range(num_e)], axis=1
    ).astype(jnp.bfloat16)
    delta = jnp.dot(hw, bflat_ref[...], preferred_element_type=jnp.float32)
    out_ref[...] = base + SCALING * delta + bias_ref[...]


def kernel(x, base_W, base_b, A, B, router_W):
    n, d_in = x.shape
    d_out = base_W.shape[0]
    e, _, r = B.shape
    tn = 512 if n % 512 == 0 else n

    xb = x.astype(jnp.bfloat16)
    w_t = base_W.T.astype(jnp.bfloat16)
    a_t = A.T.astype(jnp.bfloat16)
    r_t = router_W.T.astype(jnp.bfloat16)
    b_flat = B.transpose(0, 2, 1).reshape(e * r, d_out).astype(jnp.bfloat16)
    bias = base_b.reshape(1, d_out)

    return pl.pallas_call(
        _fused_kernel,
        grid=(n // tn,),
        in_specs=[
            pl.BlockSpec((tn, d_in), lambda i: (i, 0)),
            pl.BlockSpec((d_in, d_out), lambda i: (0, 0)),
            pl.BlockSpec((d_in, r), lambda i: (0, 0)),
            pl.BlockSpec((d_in, e), lambda i: (0, 0)),
            pl.BlockSpec((e * r, d_out), lambda i: (0, 0)),
            pl.BlockSpec((1, d_out), lambda i: (0, 0)),
        ],
        out_specs=pl.BlockSpec((tn, d_out), lambda i: (i, 0)),
        out_shape=jax.ShapeDtypeStruct((n, d_out), jnp.float32),
    )(xb, w_t, a_t, r_t, b_flat, bias)


# speedup vs baseline: 4.7387x; 1.2242x over previous
import jax
import jax.numpy as jnp
from jax.experimental import pallas as pl

SCALING = 32.0 / 64.0


def _fused_kernel(x_ref, wcat_ref, a_ref, r_ref, s_ref, t_ref,
                  bias_ref, out_ref):
    xb = x_ref[...].astype(jnp.bfloat16)
    h = jnp.dot(xb, a_ref[...], preferred_element_type=jnp.float32)
    h = jnp.maximum(h, 0.0)
    logits = jnp.dot(xb, r_ref[...], preferred_element_type=jnp.float32)
    m = jnp.max(logits, axis=-1, keepdims=True)
    p = jnp.exp(logits - m)
    wts = p / jnp.sum(p, axis=-1, keepdims=True)
    w_rep = jnp.dot(wts.astype(jnp.bfloat16), s_ref[...],
                    preferred_element_type=jnp.float32)
    h_tile = jnp.dot(h.astype(jnp.bfloat16), t_ref[...],
                     preferred_element_type=jnp.float32)
    hw = (w_rep * h_tile).astype(jnp.bfloat16)
    xcat = jnp.concatenate([xb, hw], axis=1)
    acc = jnp.dot(xcat, wcat_ref[...], preferred_element_type=jnp.float32)
    out_ref[...] = acc + bias_ref[...]


def kernel(x, base_W, base_b, A, B, router_W):
    n, d_in = x.shape
    d_out = base_W.shape[0]
    e, _, r = B.shape
    tn = 1024 if n % 1024 == 0 else n

    w_t = base_W.T.astype(jnp.bfloat16)
    a_t = A.T.astype(jnp.bfloat16)
    r_t = router_W.T.astype(jnp.bfloat16)
    b_flat = (SCALING * B.transpose(0, 2, 1).reshape(e * r, d_out)
              ).astype(jnp.bfloat16)
    w_cat = jnp.concatenate([w_t, b_flat], axis=0)
    bias = base_b.reshape(1, d_out)
    j = jnp.arange(e * r)
    s_pat = (j // r == jnp.arange(e)[:, None]).astype(jnp.bfloat16)
    t_pat = (j % r == jnp.arange(r)[:, None]).astype(jnp.bfloat16)

    return pl.pallas_call(
        _fused_kernel,
        grid=(n // tn,),
        in_specs=[
            pl.BlockSpec((tn, d_in), lambda i: (i, 0)),
            pl.BlockSpec((d_in + e * r, d_out), lambda i: (0, 0)),
            pl.BlockSpec((d_in, r), lambda i: (0, 0)),
            pl.BlockSpec((d_in, e), lambda i: (0, 0)),
            pl.BlockSpec((e, e * r), lambda i: (0, 0)),
            pl.BlockSpec((r, e * r), lambda i: (0, 0)),
            pl.BlockSpec((1, d_out), lambda i: (0, 0)),
        ],
        out_specs=pl.BlockSpec((tn, d_out), lambda i: (i, 0)),
        out_shape=jax.ShapeDtypeStruct((n, d_out), jnp.float32),
    )(x, w_cat, a_t, r_t, s_pat, t_pat, bias)
